# baseline (speedup 1.0000x reference)
import functools

import jax
import jax.numpy as jnp
from jax import lax
from jax.experimental import pallas as pl
from jax.experimental.pallas import tpu as pltpu
from jax.experimental.pallas import tpu_sc as plsc

NUM_CLASSES = 100000
FEAT_DIM = 128
BATCH = 4096
ALPHA = 0.5
HALF = NUM_CLASSES // 2
NC, NS = 2, 16
NW = NC * NS
B_PER_W = BATCH // NW
B_PER_S = BATCH // NS
ROWS_PER_S = HALF // NS
BLK = 512
NBLK = BATCH // BLK

@functools.cache
def _sc_mesh():
    return plsc.VectorSubcoreMesh(core_axis_name="c", subcore_axis_name="s")


def _gather_body(centers_hbm, targets_hbm, out_hbm, idx_v, rows_v, sem):
    c = lax.axis_index("c")
    s = lax.axis_index("s")
    base = (s * NC + c) * B_PER_W
    pltpu.sync_copy(targets_hbm.at[pl.ds(base, B_PER_W)], idx_v)
    pltpu.async_copy(centers_hbm.at[idx_v], rows_v, sem).wait()
    pltpu.sync_copy(rows_v, out_hbm.at[pl.ds(base, B_PER_W)])


@functools.cache
def _gather():
    return pl.kernel(
        _gather_body,
        out_type=jax.ShapeDtypeStruct((BATCH, FEAT_DIM), jnp.float32),
        mesh=_sc_mesh(),
        scratch_types=[
            pltpu.VMEM((B_PER_W,), jnp.int32),
            pltpu.VMEM((B_PER_W, FEAT_DIM), jnp.float32),
            pltpu.SemaphoreType.DMA,
        ],
    )


def _compute_body(tc_ref, feat_ref, tcol_ref, trow_ref, ti_ref, cfix_ref,
                  upd_ref, midx_ref, loss_ref):
    tc = tc_ref[...]
    diff = tc - feat_ref[...]
    loss_ref[...] = (jnp.sum(diff * diff) * (1.0 / (BATCH * FEAT_DIM))
                     ).reshape(1, 1)

    trow = trow_ref[...]
    tcol = tcol_ref[...]

    cfix = cfix_ref[...]
    fixrows = []
    for c in range(2):
        m = (tcol == jnp.float32(c * HALF)).astype(jnp.float32)
        segd = jnp.sum(diff * m, axis=0, keepdims=True)
        cntd = jnp.sum(m)
        fixrows.append(cfix[c:c + 1, :] - (ALPHA * segd) / (cntd + 1.0))

    diff_bf = diff.astype(jnp.bfloat16)
    ones_col = (lax.broadcasted_iota(jnp.int32, (BATCH, FEAT_DIM), 1) == 0
                ).astype(jnp.bfloat16)
    aug = jnp.concatenate([diff_bf, ones_col], axis=1)
    for k in range(NBLK):
        tcb = tcol[k * BLK:(k + 1) * BLK, :]
        e = (tcb == trow).astype(jnp.bfloat16)
        r = jnp.dot(e, aug, preferred_element_type=jnp.float32)
        seg = r[:, :FEAT_DIM]
        cnt = r[:, FEAT_DIM:FEAT_DIM + 1]
        u = tc[k * BLK:(k + 1) * BLK, :] - (ALPHA * seg) / (cnt + 1.0)
        m0 = tcb < jnp.float32(HALF)
        upd_ref[0, k * BLK:(k + 1) * BLK, :] = jnp.where(m0, u, fixrows[0])
        upd_ref[1, k * BLK:(k + 1) * BLK, :] = jnp.where(m0, fixrows[1], u)

    ti = ti_ref[...]
    midx_ref[0, :, :] = jnp.where(ti < HALF, ti, 0)
    midx_ref[1, :, :] = jnp.where(ti >= HALF, ti, HALF)


_compute = pl.pallas_call(
    _compute_body,
    out_shape=(
        jax.ShapeDtypeStruct((2, BATCH, FEAT_DIM), jnp.float32),
        jax.ShapeDtypeStruct((2, NW, B_PER_W), jnp.int32),
        jax.ShapeDtypeStruct((1, 1), jnp.float32),
    ),
)


CHUNK = 3128
CHUNK_LO = 3080
CHUNK_REM = CHUNK - CHUNK_LO


def _scatter_body(centers_hbm, upd_hbm, midx_hbm, out_hbm,
                  idx_v, rows_v, csem, ssem):
    c = lax.axis_index("c")
    s = lax.axis_index("s")
    row0 = c * HALF + s * CHUNK
    cp = pltpu.async_copy(centers_hbm.at[pl.ds(row0, CHUNK_LO)],
                          out_hbm.at[pl.ds(row0, CHUNK_LO)], csem)

    @pl.when(s < NS - 1)
    def _():
        pltpu.async_copy(centers_hbm.at[pl.ds(row0 + CHUNK_LO, CHUNK_REM)],
                         out_hbm.at[pl.ds(row0 + CHUNK_LO, CHUNK_REM)], csem)

    pltpu.sync_copy(midx_hbm.at[c], idx_v)
    pltpu.sync_copy(upd_hbm.at[c].at[pl.ds(s * B_PER_S, B_PER_S)], rows_v)
    cp.wait()

    @pl.when(s < NS - 1)
    def _():
        pltpu.make_async_copy(
            centers_hbm.at[pl.ds(row0 + CHUNK_LO, CHUNK_REM)],
            out_hbm.at[pl.ds(row0 + CHUNK_LO, CHUNK_REM)], csem).wait()

    plsc.subcore_barrier()
    s1 = pltpu.async_copy(rows_v.at[pl.ds(0, B_PER_W)],
                          out_hbm.at[idx_v.at[2 * s]], ssem)
    s2 = pltpu.async_copy(rows_v.at[pl.ds(B_PER_W, B_PER_W)],
                          out_hbm.at[idx_v.at[2 * s + 1]], ssem)
    s1.wait()
    s2.wait()


@functools.cache
def _scatter():
    return pl.kernel(
        _scatter_body,
        out_type=jax.ShapeDtypeStruct((NUM_CLASSES, FEAT_DIM), jnp.float32),
        mesh=_sc_mesh(),
        scratch_types=[
            pltpu.VMEM((NW, B_PER_W), jnp.int32),
            pltpu.VMEM((B_PER_S, FEAT_DIM), jnp.float32),
            pltpu.SemaphoreType.DMA,
            pltpu.SemaphoreType.DMA,
        ],
    )


def kernel(features, targets, centers):
    tgt = targets.astype(jnp.int32)
    tc = _gather()(centers, tgt)
    tcol = tgt.astype(jnp.float32).reshape(BATCH, 1)
    trow = tgt.astype(jnp.float32).reshape(1, BATCH)
    ti = tgt.reshape(NW, B_PER_W)
    cfix = jnp.concatenate([centers[0:1], centers[HALF:HALF + 1]], axis=0)
    upd, midx, loss = _compute(tc, features, tcol, trow, ti, cfix)
    out = _scatter()(centers, upd, midx)
    return loss[0, 0], out


# speedup vs baseline: 19.1242x; 19.1242x over previous
import functools

import jax
import jax.numpy as jnp
from jax import lax
from jax.experimental import pallas as pl
from jax.experimental.pallas import tpu as pltpu
from jax.experimental.pallas import tpu_sc as plsc

NUM_CLASSES = 100000
FEAT_DIM = 128
BATCH = 4096
ALPHA = 0.5
NC, NS = 2, 16
NW = NC * NS
B_PER_W = BATCH // NW
BLK = 512
NBLK = BATCH // BLK
COPY_BLK = 2000


@functools.cache
def _sc_mesh():
    return plsc.VectorSubcoreMesh(core_axis_name="c", subcore_axis_name="s")


def _gather_body(centers_hbm, targets_hbm, out_hbm, idx_v, rows_v, sem):
    c = lax.axis_index("c")
    s = lax.axis_index("s")
    base = (s * NC + c) * B_PER_W
    pltpu.sync_copy(targets_hbm.at[pl.ds(base, B_PER_W)], idx_v)
    pltpu.async_copy(centers_hbm.at[idx_v], rows_v, sem).wait()
    pltpu.sync_copy(rows_v, out_hbm.at[pl.ds(base, B_PER_W)])


@functools.cache
def _gather():
    return pl.kernel(
        _gather_body,
        out_type=jax.ShapeDtypeStruct((BATCH, FEAT_DIM), jnp.float32),
        mesh=_sc_mesh(),
        scratch_types=[
            pltpu.VMEM((B_PER_W,), jnp.int32),
            pltpu.VMEM((B_PER_W, FEAT_DIM), jnp.float32),
            pltpu.SemaphoreType.DMA,
        ],
    )


def _compute_body(tc_ref, feat_ref, tcol_ref, trow_ref, upd_ref, loss_ref):
    tc = tc_ref[...]
    diff = tc - feat_ref[...]
    loss_ref[...] = (jnp.sum(diff * diff) * (1.0 / (BATCH * FEAT_DIM))
                     ).reshape(1, 1)

    trow = trow_ref[...]
    tcol = tcol_ref[...]
    diff_bf = diff.astype(jnp.bfloat16)
    ones_col = (lax.broadcasted_iota(jnp.int32, (BATCH, FEAT_DIM), 1) == 0
                ).astype(jnp.bfloat16)
    aug = jnp.concatenate([diff_bf, ones_col], axis=1)
    for k in range(NBLK):
        tcb = tcol[k * BLK:(k + 1) * BLK, :]
        e = (tcb == trow).astype(jnp.bfloat16)
        r = jnp.dot(e, aug, preferred_element_type=jnp.float32)
        seg = r[:, :FEAT_DIM]
        cnt = r[:, FEAT_DIM:FEAT_DIM + 1]
        upd_ref[k * BLK:(k + 1) * BLK, :] = (
            tc[k * BLK:(k + 1) * BLK, :] - (ALPHA * seg) / (cnt + 1.0))


_compute = pl.pallas_call(
    _compute_body,
    out_shape=(
        jax.ShapeDtypeStruct((BATCH, FEAT_DIM), jnp.float32),
        jax.ShapeDtypeStruct((1, 1), jnp.float32),
    ),
)


def _copy_body(src_ref, dst_ref):
    dst_ref[...] = src_ref[...]


_copy = pl.pallas_call(
    _copy_body,
    grid=(NUM_CLASSES // COPY_BLK,),
    in_specs=[pl.BlockSpec((COPY_BLK, FEAT_DIM), lambda i: (i, 0))],
    out_specs=pl.BlockSpec((COPY_BLK, FEAT_DIM), lambda i: (i, 0)),
    out_shape=jax.ShapeDtypeStruct((NUM_CLASSES, FEAT_DIM), jnp.float32),
)


def _scatter_body(upd_hbm, tidx_hbm, table_ref, idx_v, rows_v, sem):
    c = lax.axis_index("c")
    s = lax.axis_index("s")
    wid = s * NC + c
    pltpu.sync_copy(tidx_hbm.at[wid], idx_v)
    pltpu.sync_copy(upd_hbm.at[pl.ds(wid * B_PER_W, B_PER_W)], rows_v)
    pltpu.async_copy(rows_v, table_ref.at[idx_v.at[0]], sem).wait()


@functools.cache
def _scatter():
    return pl.kernel(
        _scatter_body,
        out_type=(),
        mesh=_sc_mesh(),
        scratch_types=[
            pltpu.VMEM((1, B_PER_W), jnp.int32),
            pltpu.VMEM((B_PER_W, FEAT_DIM), jnp.float32),
            pltpu.SemaphoreType.DMA,
        ],
    )


def kernel(features, targets, centers):
    tgt = targets.astype(jnp.int32)
    tc = _gather()(centers, tgt)
    tcol = tgt.astype(jnp.float32).reshape(BATCH, 1)
    trow = tgt.astype(jnp.float32).reshape(1, BATCH)
    upd, loss = _compute(tc, features, tcol, trow)
    base = _copy(centers)
    table = jax.new_ref(base)
    _scatter()(upd, tgt.reshape(NW, 1, B_PER_W), table)
    return loss[0, 0], table[...]


# speedup vs baseline: 19.2184x; 1.0049x over previous
import functools

import jax
import jax.numpy as jnp
from jax import lax
from jax.experimental import pallas as pl
from jax.experimental.pallas import tpu as pltpu
from jax.experimental.pallas import tpu_sc as plsc

NUM_CLASSES = 100000
FEAT_DIM = 128
BATCH = 4096
ALPHA = 0.5
NC, NS = 2, 16
NW = NC * NS
B_PER_W = BATCH // NW
BLK = 512
NBLK = BATCH // BLK
COPY_BLK = 2000


@functools.cache
def _sc_mesh():
    return plsc.VectorSubcoreMesh(core_axis_name="c", subcore_axis_name="s")


def _gather_body(centers_hbm, targets_hbm, out_hbm, idx_v, rows_v, sem):
    c = lax.axis_index("c")
    s = lax.axis_index("s")
    base = (s * NC + c) * B_PER_W
    pltpu.sync_copy(targets_hbm.at[pl.ds(base, B_PER_W)], idx_v)
    pltpu.async_copy(centers_hbm.at[idx_v], rows_v, sem).wait()
    pltpu.sync_copy(rows_v, out_hbm.at[pl.ds(base, B_PER_W)])


@functools.cache
def _gather():
    return pl.kernel(
        _gather_body,
        out_type=jax.ShapeDtypeStruct((BATCH, FEAT_DIM), jnp.float32),
        mesh=_sc_mesh(),
        scratch_types=[
            pltpu.VMEM((B_PER_W,), jnp.int32),
            pltpu.VMEM((B_PER_W, FEAT_DIM), jnp.float32),
            pltpu.SemaphoreType.DMA,
        ],
    )


def _fused_body(tc_ref, feat_ref, tcol_ref, trow_ref, src_ref,
                dst_ref, upd_ref, loss_ref, aug_ref):
    i = pl.program_id(0)
    dst_ref[...] = src_ref[...]

    @pl.when(i == 0)
    def _():
        diff = tc_ref[...] - feat_ref[...]
        loss_ref[...] = (jnp.sum(diff * diff) * (1.0 / (BATCH * FEAT_DIM))
                         ).reshape(1, 1)
        aug_ref[:, :FEAT_DIM] = diff.astype(jnp.bfloat16)
        aug_ref[:, FEAT_DIM:] = (
            lax.broadcasted_iota(jnp.int32, (BATCH, FEAT_DIM), 1) == 0
        ).astype(jnp.bfloat16)

    @pl.when(i < NBLK)
    def _():
        tcb = tcol_ref[pl.ds(i * BLK, BLK), :]
        e = (tcb == trow_ref[...]).astype(jnp.bfloat16)
        r = jnp.dot(e, aug_ref[...], preferred_element_type=jnp.float32)
        seg = r[:, :FEAT_DIM]
        cnt = r[:, FEAT_DIM:FEAT_DIM + 1]
        upd_ref[pl.ds(i * BLK, BLK), :] = (
            tc_ref[pl.ds(i * BLK, BLK), :] - (ALPHA * seg) / (cnt + 1.0))


_full = lambda i: (0, 0)
_fused = pl.pallas_call(
    _fused_body,
    grid=(NUM_CLASSES // COPY_BLK,),
    in_specs=[
        pl.BlockSpec((BATCH, FEAT_DIM), _full),
        pl.BlockSpec((BATCH, FEAT_DIM), _full),
        pl.BlockSpec((BATCH, 1), _full),
        pl.BlockSpec((1, BATCH), _full),
        pl.BlockSpec((COPY_BLK, FEAT_DIM), lambda i: (i, 0)),
    ],
    out_specs=(
        pl.BlockSpec((COPY_BLK, FEAT_DIM), lambda i: (i, 0)),
        pl.BlockSpec((BATCH, FEAT_DIM), _full),
        pl.BlockSpec((1, 1), _full),
    ),
    out_shape=(
        jax.ShapeDtypeStruct((NUM_CLASSES, FEAT_DIM), jnp.float32),
        jax.ShapeDtypeStruct((BATCH, FEAT_DIM), jnp.float32),
        jax.ShapeDtypeStruct((1, 1), jnp.float32),
    ),
    scratch_shapes=[pltpu.VMEM((BATCH, 2 * FEAT_DIM), jnp.bfloat16)],
)


def _scatter_body(upd_hbm, tidx_hbm, table_ref, idx_v, rows_v, sem):
    c = lax.axis_index("c")
    s = lax.axis_index("s")
    wid = s * NC + c
    pltpu.sync_copy(tidx_hbm.at[wid], idx_v)
    pltpu.sync_copy(upd_hbm.at[pl.ds(wid * B_PER_W, B_PER_W)], rows_v)
    pltpu.async_copy(rows_v, table_ref.at[idx_v.at[0]], sem).wait()


@functools.cache
def _scatter():
    return pl.kernel(
        _scatter_body,
        out_type=(),
        mesh=_sc_mesh(),
        scratch_types=[
            pltpu.VMEM((1, B_PER_W), jnp.int32),
            pltpu.VMEM((B_PER_W, FEAT_DIM), jnp.float32),
            pltpu.SemaphoreType.DMA,
        ],
    )


def kernel(features, targets, centers):
    tgt = targets.astype(jnp.int32)
    tc = _gather()(centers, tgt)
    tcol = tgt.astype(jnp.float32).reshape(BATCH, 1)
    trow = tgt.astype(jnp.float32).reshape(1, BATCH)
    base, upd, loss = _fused(tc, features, tcol, trow, centers)
    table = jax.new_ref(base)
    _scatter()(upd, tgt.reshape(NW, 1, B_PER_W), table)
    return loss[0, 0], table[...]
